# baseline (speedup 1.0000x reference)
import functools

import numpy as np
import jax
import jax.numpy as jnp
from jax.experimental import pallas as pl
from jax.experimental.pallas import tpu as pltpu

_HEADS = 8
_DIM_HEAD = 64
_EDGE = 33
_HW = _EDGE * _EDGE
_NTOK = 129
_NPAD = 136
_DIM = 256
_INNER = _HEADS * _DIM_HEAD
_G = 8


def _build_spiral(edge):
    mid = int(edge / 2)
    rings = []
    last_idx = None
    for i in range(1, mid + 1):
        a1, a2, b1, b2 = mid - i, mid + i, mid - i, mid + i
        e1, e2, e3, e4 = 0, edge, 0, edge
        pos = []
        l1, l2 = a1, b1 - 1
        direction = [0, 1]
        start = 0
        while True:
            l1 += direction[0]; l2 += direction[1]
            if l1 == a1 and l2 == b1:
                if start == 1:
                    break
                start = 1
            if l1 > a2:
                direction = [0, -1]; l1 -= 1; continue
            elif l2 < b1:
                direction = [-1, 0]; l2 += 1; continue
            elif l2 > b2:
                direction = [1, 0]; l2 -= 1; continue
            if l1 < e1 or l1 > e2 or l2 < e3 or l2 > e4:
                continue
            pos.append((l1, l2))
        idx, sA, sB = [], [], []
        for (p1, p2) in pos:
            m1 = 0 if p1 == mid else (-1 if p1 > mid else 1)
            m2 = 0 if p2 == mid else (-1 if p2 > mid else 1)
            idx.append(p1 * edge + p2)
            if abs(p1 - mid) > abs(p2 - mid):
                sA.append((p1 + m1) * edge + p2)
                sB.append((p1 + m1) * edge + p2 + m2)
            elif abs(p1 - mid) < abs(p2 - mid):
                sA.append(p1 * edge + p2 + m2)
                sB.append((p1 + m1) * edge + p2 + m2)
            else:
                sA.append((p1 + m1) * edge + p2 + m2)
                sB.append((p1 + m1) * edge + p2 + m2)
        rings.append((np.asarray(idx), np.asarray(sA), np.asarray(sB)))
        last_idx = np.asarray(idx)
    return rings, last_idx


@functools.lru_cache(maxsize=None)
def _spiral_matrix(edge):
    rings, last_idx = _build_spiral(edge)
    n = edge * edge
    A = np.eye(n, dtype=np.float64)
    for idx, sA, sB in rings:
        A[idx] = A[idx] + 0.5 * (A[sA] + A[sB])
    return np.ascontiguousarray(A[last_idx]).astype(np.float32)


@functools.lru_cache(maxsize=None)
def _sinusoid_table(n_position, d_hid, n_pad):
    pos = np.arange(n_position)[:, None].astype(np.float64)
    j = np.arange(d_hid)[None, :]
    angle = pos / np.power(10000.0, 2 * (j // 2) / d_hid)
    table = np.zeros((n_position, d_hid))
    table[:, 0::2] = np.sin(angle[:, 0::2])
    table[:, 1::2] = np.cos(angle[:, 1::2])
    out = np.zeros((n_pad, d_hid))
    out[:n_position - 1] = table[1:]
    out[n_position - 1] = table[0]
    return out.astype(np.float32)


_BF = jnp.bfloat16


def _mm(a, b):
    return jax.lax.dot_general(
        a.astype(_BF), b.astype(_BF), (((1,), (0,)), ((), ())),
        preferred_element_type=jnp.float32)


def _mm_nt(a, b):
    return jax.lax.dot_general(
        a.astype(_BF), b.astype(_BF), (((1,), (1,)), ((), ())),
        preferred_element_type=jnp.float32)


def _mmb(a, b):
    return _mm(a, b).astype(_BF)


def _ln(x, g, b, eps=1e-5):
    mu = jnp.mean(x, axis=-1, keepdims=True)
    var = jnp.mean((x - mu) ** 2, axis=-1, keepdims=True)
    return (x - mu) / jnp.sqrt(var + eps) * g + b


def _ln_mx(x, g, b, u, eps=1e-5):
    mu = _mm(x, u)[:, 0:1]
    m2 = _mm(x * x, u)[:, 0:1]
    var = m2 - mu * mu
    return (x - mu) / jnp.sqrt(var + eps) * g + b


def _vit_kernel(x_ref, Wsp_ref, pe_ref, eW_ref, eb_ref, cls_ref,
                Wqkv_ref, Wo_ref, bo_ref, g1_ref, be1_ref, W1_ref, b1_ref,
                W2_ref, b2_ref, g2_ref, be2_ref, hg_ref, hb_ref, Wh_ref,
                bh_ref, out_ref, t_scr, o_scr, p_scr, va_scr):
    depth = Wqkv_ref.shape[0]
    f32 = jnp.float32
    scale = _DIM_HEAD ** -0.5

    va_scr[...] = jnp.where(
        jax.lax.broadcasted_iota(jnp.int32, va_scr.shape, 1) == _DIM_HEAD,
        1.0, 0.0).astype(_BF)

    for g in range(_G):
        xg = x_ref[g]
        tokT = _mm(xg, Wsp_ref[...])
        tok = tokT.T
        emb = _mm(tok, eW_ref[...]) + eb_ref[...]
        r0 = g * _NPAD
        nt = _NTOK - 1
        t_scr[r0:r0 + nt, :] = emb + pe_ref[0:nt, :]
        t_scr[r0 + nt:r0 + _NTOK, :] = cls_ref[...] + pe_ref[nt:_NTOK, :]
        t_scr[r0 + _NTOK:r0 + _NPAD, :] = jnp.zeros(
            (_NPAD - _NTOK, _DIM), f32)
    t = t_scr[...]

    colbias = jnp.where(
        jax.lax.broadcasted_iota(jnp.int32, (1, _NPAD), 1) < _NTOK,
        0.0, -1e30)

    for l in range(depth):
        h = _ln(t, g1_ref[l], be1_ref[l])
        qkv = _mm(h, Wqkv_ref[l])
        for g in range(_G):
            r0 = g * _NPAD
            for hd in range(_HEADS):
                c0 = hd * _DIM_HEAD
                qh = qkv[r0:r0 + _NPAD, c0:c0 + _DIM_HEAD] * scale
                kh = qkv[r0:r0 + _NPAD, _INNER + c0:_INNER + c0 + _DIM_HEAD]
                vh = qkv[r0:r0 + _NPAD,
                         2 * _INNER + c0:2 * _INNER + c0 + _DIM_HEAD]
                s = _mm_nt(qh, kh) + colbias
                e = jnp.exp(s)
                va_scr[:, :_DIM_HEAD] = vh.astype(_BF)
                oa = _mm(e, va_scr[...])
                o_scr[r0:r0 + _NPAD, c0:c0 + _DIM_HEAD] = (
                    oa[:, :_DIM_HEAD] / oa[:, _DIM_HEAD:_DIM_HEAD + 1])
        t = t + _mm(o_scr[...], Wo_ref[l]) + bo_ref[l]
        h2 = _ln(t, g2_ref[l], be2_ref[l])
        a = _mm(h2, W1_ref[l]) + b1_ref[l]
        z = jnp.abs(a) * (2.0 ** -0.5)
        tt = 1.0 / (1.0 + 0.3275911 * z)
        poly = tt * (0.254829592 + tt * (-0.284496736 + tt * (
            1.421413741 + tt * (-1.453152027 + tt * 1.061405429))))
        erfv = 1.0 - poly * jnp.exp(-(z * z))
        erfs = jnp.where(a < 0.0, -erfv, erfv)
        a = a * 0.5 * (1.0 + erfs)
        t = t + _mm(a, W2_ref[l]) + b2_ref[l]

    rmask = jax.lax.broadcasted_iota(
        jnp.int32, (_NPAD, _DIM), 0) < _NTOK
    for g in range(_G):
        r0 = g * _NPAD
        p_scr[g:g + 1, :] = jnp.sum(
            jnp.where(rmask, t[r0:r0 + _NPAD, :], 0.0),
            axis=0, keepdims=True) / float(_NTOK)
    pooled = _ln(p_scr[...], hg_ref[...], hb_ref[...])
    res = jnp.dot(pooled, Wh_ref[...],
                  preferred_element_type=f32) + bh_ref[...]
    out_ref[...] = res.reshape(_G, 1, res.shape[-1])


def kernel(x, embed_W, embed_b, cls_token, Wqkv, Wo, bo, ln1_g, ln1_b,
           W1, b1, W2, b2, ln2_g, ln2_b, head_g, head_b, Whead, bhead):
    B, C, S, _ = x.shape
    depth = Wqkv.shape[0]
    dim = embed_W.shape[1]
    mlp_dim = W1.shape[2]
    ncls = Whead.shape[1]
    ncls_pad = 128

    x_r = x.reshape(B, C, S * S)
    Wsp = jnp.asarray(_spiral_matrix(S).T)
    pe = jnp.asarray(_sinusoid_table(_NTOK, dim, _NPAD))
    Whp = jnp.pad(Whead, ((0, 0), (0, ncls_pad - ncls)))
    bhp = jnp.pad(bhead, (0, ncls_pad - ncls)).reshape(1, ncls_pad)
    scale = _DIM_HEAD ** -0.5
    Wqkv_s = jnp.concatenate(
        [Wqkv[:, :, :_INNER] * scale, Wqkv[:, :, _INNER:]], axis=2)
    uvec = jnp.where(
        jnp.arange(ncls_pad)[None, :] == 0, 1.0 / dim, 0.0
    ) * jnp.ones((dim, 1))
    uvec = uvec.astype(jnp.float32)

    const2 = lambda i: (0, 0)
    const3 = lambda i: (0, 0, 0)

    out = pl.pallas_call(
        _vit_kernel,
        grid=(B // _G,),
        in_specs=[
            pl.BlockSpec((_G, C, S * S), lambda i: (i, 0, 0)),
            pl.BlockSpec(Wsp.shape, const2),
            pl.BlockSpec((_NPAD, dim), const2),
            pl.BlockSpec((C, dim), const2),
            pl.BlockSpec((1, dim), const2),
            pl.BlockSpec((1, dim), const2),
            pl.BlockSpec((depth, dim, 3 * _INNER), const3),
            pl.BlockSpec((depth, _INNER, dim), const3),
            pl.BlockSpec((depth, 1, dim), const3),
            pl.BlockSpec((depth, 1, dim), const3),
            pl.BlockSpec((depth, 1, dim), const3),
            pl.BlockSpec((depth, dim, mlp_dim), const3),
            pl.BlockSpec((depth, 1, mlp_dim), const3),
            pl.BlockSpec((depth, mlp_dim, dim), const3),
            pl.BlockSpec((depth, 1, dim), const3),
            pl.BlockSpec((depth, 1, dim), const3),
            pl.BlockSpec((depth, 1, dim), const3),
            pl.BlockSpec((1, dim), const2),
            pl.BlockSpec((1, dim), const2),
            pl.BlockSpec((dim, ncls_pad), const2),
            pl.BlockSpec((1, ncls_pad), const2),
        ],
        out_specs=pl.BlockSpec((_G, 1, ncls_pad), lambda i: (i, 0, 0)),
        out_shape=jax.ShapeDtypeStruct((B, 1, ncls_pad), jnp.float32),
        scratch_shapes=[
            pltpu.VMEM((_G * _NPAD, dim), jnp.float32),
            pltpu.VMEM((_G * _NPAD, _INNER), jnp.float32),
            pltpu.VMEM((_G, dim), jnp.float32),
            pltpu.VMEM((_NPAD, 2 * _DIM_HEAD), _BF),
        ],
        compiler_params=pltpu.CompilerParams(
            dimension_semantics=("parallel",),
        ),
    )(
        x_r, Wsp.astype(_BF), pe, embed_W.astype(_BF),
        embed_b.reshape(1, dim),
        cls_token.reshape(1, dim), Wqkv.astype(_BF), Wo.astype(_BF),
        bo.reshape(depth, 1, dim),
        ln1_g.reshape(depth, 1, dim), ln1_b.reshape(depth, 1, dim),
        W1.astype(_BF), b1.reshape(depth, 1, mlp_dim), W2.astype(_BF),
        b2.reshape(depth, 1, dim),
        ln2_g.reshape(depth, 1, dim), ln2_b.reshape(depth, 1, dim),
        head_g.reshape(1, dim), head_b.reshape(1, dim), Whp, bhp,
    )
    return out.reshape(B, ncls_pad)[:, :ncls]


# speedup vs baseline: 1.1717x; 1.1717x over previous
import functools

import numpy as np
import jax
import jax.numpy as jnp
from jax.experimental import pallas as pl
from jax.experimental.pallas import tpu as pltpu

_HEADS = 8
_DIM_HEAD = 64
_EDGE = 33
_HW = _EDGE * _EDGE
_NTOK = 129
_NPAD = 136
_DIM = 256
_INNER = _HEADS * _DIM_HEAD
_G = 8


def _build_spiral(edge):
    mid = int(edge / 2)
    rings = []
    last_idx = None
    for i in range(1, mid + 1):
        a1, a2, b1, b2 = mid - i, mid + i, mid - i, mid + i
        e1, e2, e3, e4 = 0, edge, 0, edge
        pos = []
        l1, l2 = a1, b1 - 1
        direction = [0, 1]
        start = 0
        while True:
            l1 += direction[0]; l2 += direction[1]
            if l1 == a1 and l2 == b1:
                if start == 1:
                    break
                start = 1
            if l1 > a2:
                direction = [0, -1]; l1 -= 1; continue
            elif l2 < b1:
                direction = [-1, 0]; l2 += 1; continue
            elif l2 > b2:
                direction = [1, 0]; l2 -= 1; continue
            if l1 < e1 or l1 > e2 or l2 < e3 or l2 > e4:
                continue
            pos.append((l1, l2))
        idx, sA, sB = [], [], []
        for (p1, p2) in pos:
            m1 = 0 if p1 == mid else (-1 if p1 > mid else 1)
            m2 = 0 if p2 == mid else (-1 if p2 > mid else 1)
            idx.append(p1 * edge + p2)
            if abs(p1 - mid) > abs(p2 - mid):
                sA.append((p1 + m1) * edge + p2)
                sB.append((p1 + m1) * edge + p2 + m2)
            elif abs(p1 - mid) < abs(p2 - mid):
                sA.append(p1 * edge + p2 + m2)
                sB.append((p1 + m1) * edge + p2 + m2)
            else:
                sA.append((p1 + m1) * edge + p2 + m2)
                sB.append((p1 + m1) * edge + p2 + m2)
        rings.append((np.asarray(idx), np.asarray(sA), np.asarray(sB)))
        last_idx = np.asarray(idx)
    return rings, last_idx


@functools.lru_cache(maxsize=None)
def _spiral_matrix(edge):
    rings, last_idx = _build_spiral(edge)
    n = edge * edge
    A = np.eye(n, dtype=np.float64)
    for idx, sA, sB in rings:
        A[idx] = A[idx] + 0.5 * (A[sA] + A[sB])
    return np.ascontiguousarray(A[last_idx]).astype(np.float32)


@functools.lru_cache(maxsize=None)
def _sinusoid_table(n_position, d_hid, n_pad):
    pos = np.arange(n_position)[:, None].astype(np.float64)
    j = np.arange(d_hid)[None, :]
    angle = pos / np.power(10000.0, 2 * (j // 2) / d_hid)
    table = np.zeros((n_position, d_hid))
    table[:, 0::2] = np.sin(angle[:, 0::2])
    table[:, 1::2] = np.cos(angle[:, 1::2])
    out = np.zeros((n_pad, d_hid))
    out[:n_position - 1] = table[1:]
    out[n_position - 1] = table[0]
    return out.astype(np.float32)


_BF = jnp.bfloat16


def _mm(a, b):
    return jax.lax.dot_general(
        a.astype(_BF), b.astype(_BF), (((1,), (0,)), ((), ())),
        preferred_element_type=jnp.float32)


def _mm_nt(a, b):
    return jax.lax.dot_general(
        a.astype(_BF), b.astype(_BF), (((1,), (1,)), ((), ())),
        preferred_element_type=jnp.float32)


def _mmb(a, b):
    return _mm(a, b).astype(_BF)


def _ln(x, g, b, eps=1e-5):
    mu = jnp.mean(x, axis=-1, keepdims=True)
    var = jnp.mean((x - mu) ** 2, axis=-1, keepdims=True)
    return (x - mu) * jax.lax.rsqrt(var + eps) * g + b


def _ln_mx(x, g, b, u, eps=1e-5):
    mu = _mm(x, u)[:, 0:1]
    m2 = _mm(x * x, u)[:, 0:1]
    var = m2 - mu * mu
    return (x - mu) / jnp.sqrt(var + eps) * g + b


def _vit_kernel(x_ref, Wsp_ref, pe_ref, eW_ref, eb_ref, cls_ref,
                Wqkv_ref, Wo_ref, bo_ref, g1_ref, be1_ref, W1_ref, b1_ref,
                W2_ref, b2_ref, g2_ref, be2_ref, hg_ref, hb_ref, Wh_ref,
                bh_ref, out_ref, t_scr, o_scr, p_scr, va_scr):
    depth = Wqkv_ref.shape[0]
    f32 = jnp.float32
    scale = _DIM_HEAD ** -0.5

    va_scr[...] = jnp.where(
        jax.lax.broadcasted_iota(jnp.int32, va_scr.shape, 1) == _DIM_HEAD,
        1.0, 0.0).astype(_BF)

    for g in range(_G):
        xg = x_ref[g]
        tokT = _mm(xg, Wsp_ref[...])
        tok = tokT.T
        emb = _mm(tok, eW_ref[...]) + eb_ref[...]
        r0 = g * _NPAD
        nt = _NTOK - 1
        t_scr[r0:r0 + nt, :] = emb + pe_ref[0:nt, :]
        t_scr[r0 + nt:r0 + _NTOK, :] = cls_ref[...] + pe_ref[nt:_NTOK, :]
        t_scr[r0 + _NTOK:r0 + _NPAD, :] = jnp.zeros(
            (_NPAD - _NTOK, _DIM), f32)
    t = t_scr[...]

    colbias = jnp.where(
        jax.lax.broadcasted_iota(jnp.int32, (1, _NPAD), 1) < _NTOK,
        0.0, -1e30)

    for l in range(depth):
        h = _ln(t, g1_ref[l], be1_ref[l])
        qkv = _mm(h, Wqkv_ref[l])
        for g in range(_G):
            r0 = g * _NPAD
            for hd in range(_HEADS):
                c0 = hd * _DIM_HEAD
                qh = qkv[r0:r0 + _NPAD, c0:c0 + _DIM_HEAD] * scale
                kh = qkv[r0:r0 + _NPAD, _INNER + c0:_INNER + c0 + _DIM_HEAD]
                vh = qkv[r0:r0 + _NPAD,
                         2 * _INNER + c0:2 * _INNER + c0 + _DIM_HEAD]
                s = _mm_nt(qh, kh) + colbias
                e = jnp.exp(s)
                va_scr[:, :_DIM_HEAD] = vh.astype(_BF)
                oa = _mm(e, va_scr[...])
                o_scr[r0:r0 + _NPAD, c0:c0 + _DIM_HEAD] = (
                    oa[:, :_DIM_HEAD] / oa[:, _DIM_HEAD:_DIM_HEAD + 1])
        t = t + _mm(o_scr[...], Wo_ref[l]) + bo_ref[l]
        h2 = _ln(t, g2_ref[l], be2_ref[l])
        a = _mm(h2, W1_ref[l]) + b1_ref[l]
        a = a * 0.5 * (1.0 + jax.lax.erf(a * (2.0 ** -0.5)))
        t = t + _mm(a, W2_ref[l]) + b2_ref[l]

    rmask = jax.lax.broadcasted_iota(
        jnp.int32, (_NPAD, _DIM), 0) < _NTOK
    for g in range(_G):
        r0 = g * _NPAD
        p_scr[g:g + 1, :] = jnp.sum(
            jnp.where(rmask, t[r0:r0 + _NPAD, :], 0.0),
            axis=0, keepdims=True) / float(_NTOK)
    pooled = _ln(p_scr[...], hg_ref[...], hb_ref[...])
    res = jnp.dot(pooled, Wh_ref[...],
                  preferred_element_type=f32) + bh_ref[...]
    out_ref[...] = res.reshape(_G, 1, res.shape[-1])


def kernel(x, embed_W, embed_b, cls_token, Wqkv, Wo, bo, ln1_g, ln1_b,
           W1, b1, W2, b2, ln2_g, ln2_b, head_g, head_b, Whead, bhead):
    B, C, S, _ = x.shape
    depth = Wqkv.shape[0]
    dim = embed_W.shape[1]
    mlp_dim = W1.shape[2]
    ncls = Whead.shape[1]
    ncls_pad = 128

    x_r = x.reshape(B, C, S * S)
    Wsp = jnp.asarray(_spiral_matrix(S).T)
    pe = jnp.asarray(_sinusoid_table(_NTOK, dim, _NPAD))
    Whp = jnp.pad(Whead, ((0, 0), (0, ncls_pad - ncls)))
    bhp = jnp.pad(bhead, (0, ncls_pad - ncls)).reshape(1, ncls_pad)
    scale = _DIM_HEAD ** -0.5
    Wqkv_s = jnp.concatenate(
        [Wqkv[:, :, :_INNER] * scale, Wqkv[:, :, _INNER:]], axis=2)
    uvec = jnp.where(
        jnp.arange(ncls_pad)[None, :] == 0, 1.0 / dim, 0.0
    ) * jnp.ones((dim, 1))
    uvec = uvec.astype(jnp.float32)

    const2 = lambda i: (0, 0)
    const3 = lambda i: (0, 0, 0)

    out = pl.pallas_call(
        _vit_kernel,
        grid=(B // _G,),
        in_specs=[
            pl.BlockSpec((_G, C, S * S), lambda i: (i, 0, 0)),
            pl.BlockSpec(Wsp.shape, const2),
            pl.BlockSpec((_NPAD, dim), const2),
            pl.BlockSpec((C, dim), const2),
            pl.BlockSpec((1, dim), const2),
            pl.BlockSpec((1, dim), const2),
            pl.BlockSpec((depth, dim, 3 * _INNER), const3),
            pl.BlockSpec((depth, _INNER, dim), const3),
            pl.BlockSpec((depth, 1, dim), const3),
            pl.BlockSpec((depth, 1, dim), const3),
            pl.BlockSpec((depth, 1, dim), const3),
            pl.BlockSpec((depth, dim, mlp_dim), const3),
            pl.BlockSpec((depth, 1, mlp_dim), const3),
            pl.BlockSpec((depth, mlp_dim, dim), const3),
            pl.BlockSpec((depth, 1, dim), const3),
            pl.BlockSpec((depth, 1, dim), const3),
            pl.BlockSpec((depth, 1, dim), const3),
            pl.BlockSpec((1, dim), const2),
            pl.BlockSpec((1, dim), const2),
            pl.BlockSpec((dim, ncls_pad), const2),
            pl.BlockSpec((1, ncls_pad), const2),
        ],
        out_specs=pl.BlockSpec((_G, 1, ncls_pad), lambda i: (i, 0, 0)),
        out_shape=jax.ShapeDtypeStruct((B, 1, ncls_pad), jnp.float32),
        scratch_shapes=[
            pltpu.VMEM((_G * _NPAD, dim), jnp.float32),
            pltpu.VMEM((_G * _NPAD, _INNER), jnp.float32),
            pltpu.VMEM((_G, dim), jnp.float32),
            pltpu.VMEM((_NPAD, 2 * _DIM_HEAD), _BF),
        ],
        compiler_params=pltpu.CompilerParams(
            dimension_semantics=("parallel",),
        ),
    )(
        x_r, Wsp.astype(_BF), pe, embed_W.astype(_BF),
        embed_b.reshape(1, dim),
        cls_token.reshape(1, dim), Wqkv.astype(_BF), Wo.astype(_BF),
        bo.reshape(depth, 1, dim),
        ln1_g.reshape(depth, 1, dim), ln1_b.reshape(depth, 1, dim),
        W1.astype(_BF), b1.reshape(depth, 1, mlp_dim), W2.astype(_BF),
        b2.reshape(depth, 1, dim),
        ln2_g.reshape(depth, 1, dim), ln2_b.reshape(depth, 1, dim),
        head_g.reshape(1, dim), head_b.reshape(1, dim), Whp, bhp,
    )
    return out.reshape(B, ncls_pad)[:, :ncls]
